# baseline (speedup 1.0000x reference)
import functools

import jax
import jax.numpy as jnp
from jax import lax
from jax.experimental import pallas as pl
from jax.experimental.pallas import tpu as pltpu
from jax.experimental.pallas import tpu_sc as plsc

_K = 16
_TILE = 512
_G = 8


def _dg(a, b, dims):
    return jax.lax.dot_general(a, b, (dims, ((), ())),
                               preferred_element_type=jnp.float32)


def _knn_kernel(p1_ref, p2_ref, idx_ref, w_ref, *, n1):
    b = pl.program_id(0)
    p1 = p1_ref[0]
    p2 = p2_ref[0]
    sq1 = jnp.sum(p1 * p1, axis=1, keepdims=True)
    sq2 = jnp.sum(p2 * p2, axis=0, keepdims=True)
    d2 = sq1 + sq2 - 2.0 * _dg(p1, p2, ((1,), (0,)))

    ri = lax.broadcasted_iota(jnp.int32, d2.shape, 0) + b * n1
    big = jnp.int32(2 ** 30)
    inf = jnp.float32(jnp.inf)
    d2s = d2
    ws, ids = [], []
    for _ in range(_K):
        m = jnp.min(d2s, axis=0, keepdims=True)
        hit = d2s <= m
        ids.append(jnp.min(jnp.where(hit, ri, big), axis=0, keepdims=True))
        d2s = jnp.where(hit, inf, d2s)
        ws.append(1.0 / (jnp.sqrt(jnp.maximum(m, 0.0)) + 1e-10))
    wmat = jnp.concatenate(ws, axis=0)
    wmat = wmat / jnp.sum(wmat, axis=0, keepdims=True)
    idx_ref[0] = jnp.concatenate(ids, axis=0)
    w_ref[0] = wmat


def _sc_combine_factory(m_rows, c_ch, mw):
    ng = mw // _G

    @functools.partial(
        pl.kernel,
        mesh=plsc.VectorSubcoreMesh(core_axis_name="c", subcore_axis_name="s"),
        out_type=jax.ShapeDtypeStruct((m_rows, c_ch), jnp.float32),
        scratch_types=[
            pltpu.VMEM((_G * _K,), jnp.int32),
            pltpu.VMEM((_G * _K,), jnp.float32),
            pltpu.VMEM((_G * _K, c_ch), jnp.float32),
            pltpu.VMEM((_G, c_ch), jnp.float32),
            pltpu.SemaphoreType.DMA,
        ],
    )
    def _sc_combine(f1_hbm, idx_hbm, w_hbm, out_hbm,
                    idx_v, w_v, rows_v, acc_v, sem):
        wid = lax.axis_index("s") * 2 + lax.axis_index("c")

        def body(g, _):
            base = wid * mw + g * _G
            pltpu.sync_copy(idx_hbm.at[pl.ds(base * _K, _G * _K)], idx_v)
            pltpu.sync_copy(w_hbm.at[pl.ds(base * _K, _G * _K)], w_v)
            pltpu.async_copy(f1_hbm.at[idx_v], rows_v, sem).wait()
            nc = c_ch // 16
            for q in range(_G):
                accs = [jnp.zeros((16,), jnp.float32) for _ in range(nc)]
                wvec = w_v[pl.ds(q * _K, _K)]
                for i in range(_K):
                    wb = lax.gather(
                        wvec, jnp.full((16, 1), i, jnp.int32),
                        lax.GatherDimensionNumbers(
                            offset_dims=(), collapsed_slice_dims=(0,),
                            start_index_map=(0,)),
                        (1,), mode=lax.GatherScatterMode.PROMISE_IN_BOUNDS)
                    row = q * _K + i
                    for cc in range(nc):
                        accs[cc] = accs[cc] + wb * rows_v[row, pl.ds(cc * 16, 16)]
                for cc in range(nc):
                    acc_v[q, pl.ds(cc * 16, 16)] = accs[cc]
            pltpu.sync_copy(acc_v, out_hbm.at[pl.ds(base, _G)])
            return _

        lax.fori_loop(0, ng, body, None)

    return _sc_combine


def _l0_kernel(nf_ref, f2_ref, w0_ref, b0_ref, y_ref, s_ref, ss_ref):
    b = pl.program_id(0)
    i = pl.program_id(1)
    nf = nf_ref[...]
    f2 = f2_ref[0]
    w0 = w0_ref[...]
    c = f2.shape[0]
    y = (_dg(w0[:, :c], nf, ((1,), (1,)))
         + _dg(w0[:, c:], f2, ((1,), (0,)))
         + b0_ref[...])
    y_ref[0] = y

    @pl.when(jnp.logical_and(b == 0, i == 0))
    def _():
        s_ref[...] = jnp.zeros_like(s_ref)
        ss_ref[...] = jnp.zeros_like(ss_ref)

    s_ref[...] += jnp.sum(y, axis=1, keepdims=True)
    ss_ref[...] += jnp.sum(y * y, axis=1, keepdims=True)


def _bn_mlp_kernel(y_ref, s_ref, ss_ref, g_ref, be_ref, w_ref, b_ref,
                   o_ref, s2_ref, ss2_ref, *, ntot):
    b = pl.program_id(0)
    i = pl.program_id(1)
    mean = s_ref[...] / ntot
    var = ss_ref[...] / ntot - mean * mean
    x = (y_ref[0] - mean) * jax.lax.rsqrt(var + 1e-3) * g_ref[...] + be_ref[...]
    h = jnp.maximum(x, 0.0)
    y2 = _dg(w_ref[...], h, ((1,), (0,))) + b_ref[...]

    o_ref[0] = y2

    @pl.when(jnp.logical_and(b == 0, i == 0))
    def _():
        s2_ref[...] = jnp.zeros_like(s2_ref)
        ss2_ref[...] = jnp.zeros_like(ss2_ref)

    s2_ref[...] += jnp.sum(y2, axis=1, keepdims=True)
    ss2_ref[...] += jnp.sum(y2 * y2, axis=1, keepdims=True)


def _bn_out_kernel(y_ref, s_ref, ss_ref, g_ref, be_ref, o_ref, *, ntot):
    mean = s_ref[...] / ntot
    var = ss_ref[...] / ntot - mean * mean
    x = (y_ref[0] - mean) * jax.lax.rsqrt(var + 1e-3) * g_ref[...] + be_ref[...]
    o_ref[0] = jnp.maximum(x, 0.0)


def kernel(points1, points2, features1, features2,
           W0, b0, g0, beta0, W1, b1, g1, beta1, W2, b2, g2, beta2):
    f32 = jnp.float32
    B, _, N1 = points1.shape
    N2 = points2.shape[2]
    C = features1.shape[1]
    OUT0 = W0.shape[0]
    OUT1 = W1.shape[0]
    OUT2 = W2.shape[0]
    T = min(_TILE, N2)
    NT = N2 // T
    M = B * N2

    p1t = jnp.concatenate(
        [points1, jnp.zeros((B, 8 - points1.shape[1], N1), f32)],
        axis=1).transpose(0, 2, 1)
    p2p = jnp.concatenate(
        [points2, jnp.zeros((B, 8 - points2.shape[1], N2), f32)],
        axis=1)

    col = lambda v: v.reshape(-1, 1)

    idx, wts = pl.pallas_call(
        functools.partial(_knn_kernel, n1=N1),
        grid=(B, NT),
        in_specs=[
            pl.BlockSpec((1, N1, 8), lambda b, i: (b, 0, 0)),
            pl.BlockSpec((1, 8, T), lambda b, i: (b, 0, i)),
        ],
        out_specs=[
            pl.BlockSpec((1, _K, T), lambda b, i: (b, 0, i)),
            pl.BlockSpec((1, _K, T), lambda b, i: (b, 0, i)),
        ],
        out_shape=[
            jax.ShapeDtypeStruct((B, _K, N2), jnp.int32),
            jax.ShapeDtypeStruct((B, _K, N2), f32),
        ],
    )(p1t, p2p)

    f1flat = features1.transpose(0, 2, 1).reshape(B * N1, C)
    gidx = idx.transpose(0, 2, 1).reshape(M * _K)
    gw = wts.transpose(0, 2, 1).reshape(M * _K)
    mw = M // 32
    newf = _sc_combine_factory(M, C, mw)(f1flat, gidx, gw)

    y0, s0, ss0 = pl.pallas_call(
        _l0_kernel,
        grid=(B, NT),
        in_specs=[
            pl.BlockSpec((T, C), lambda b, i: (b * NT + i, 0)),
            pl.BlockSpec((1, C, T), lambda b, i: (b, 0, i)),
            pl.BlockSpec((OUT0, 2 * C), lambda b, i: (0, 0)),
            pl.BlockSpec((OUT0, 1), lambda b, i: (0, 0)),
        ],
        out_specs=[
            pl.BlockSpec((1, OUT0, T), lambda b, i: (b, 0, i)),
            pl.BlockSpec((OUT0, 1), lambda b, i: (0, 0)),
            pl.BlockSpec((OUT0, 1), lambda b, i: (0, 0)),
        ],
        out_shape=[
            jax.ShapeDtypeStruct((B, OUT0, N2), f32),
            jax.ShapeDtypeStruct((OUT0, 1), f32),
            jax.ShapeDtypeStruct((OUT0, 1), f32),
        ],
    )(newf, features2, W0, col(b0))

    def _layer(y, s, ss, g, be, W, bias, cin, cout):
        return pl.pallas_call(
            functools.partial(_bn_mlp_kernel, ntot=float(M)),
            grid=(B, NT),
            in_specs=[
                pl.BlockSpec((1, cin, T), lambda b, i: (b, 0, i)),
                pl.BlockSpec((cin, 1), lambda b, i: (0, 0)),
                pl.BlockSpec((cin, 1), lambda b, i: (0, 0)),
                pl.BlockSpec((cin, 1), lambda b, i: (0, 0)),
                pl.BlockSpec((cin, 1), lambda b, i: (0, 0)),
                pl.BlockSpec((cout, cin), lambda b, i: (0, 0)),
                pl.BlockSpec((cout, 1), lambda b, i: (0, 0)),
            ],
            out_specs=[
                pl.BlockSpec((1, cout, T), lambda b, i: (b, 0, i)),
                pl.BlockSpec((cout, 1), lambda b, i: (0, 0)),
                pl.BlockSpec((cout, 1), lambda b, i: (0, 0)),
            ],
            out_shape=[
                jax.ShapeDtypeStruct((B, cout, N2), f32),
                jax.ShapeDtypeStruct((cout, 1), f32),
                jax.ShapeDtypeStruct((cout, 1), f32),
            ],
        )(y, s, ss, col(g), col(be), W, col(bias))

    y1, s1, ss1 = _layer(y0, s0, ss0, g0, beta0, W1, b1, OUT0, OUT1)
    y2, s2, ss2 = _layer(y1, s1, ss1, g1, beta1, W2, b2, OUT1, OUT2)

    (out,) = pl.pallas_call(
        functools.partial(_bn_out_kernel, ntot=float(M)),
        grid=(B, NT),
        in_specs=[
            pl.BlockSpec((1, OUT2, T), lambda b, i: (b, 0, i)),
            pl.BlockSpec((OUT2, 1), lambda b, i: (0, 0)),
            pl.BlockSpec((OUT2, 1), lambda b, i: (0, 0)),
            pl.BlockSpec((OUT2, 1), lambda b, i: (0, 0)),
            pl.BlockSpec((OUT2, 1), lambda b, i: (0, 0)),
        ],
        out_specs=[pl.BlockSpec((1, OUT2, T), lambda b, i: (b, 0, i))],
        out_shape=[jax.ShapeDtypeStruct((B, OUT2, N2), f32)],
    )(y2, s2, ss2, col(g2), col(beta2))

    return out


# speedup vs baseline: 2.7726x; 2.7726x over previous
import functools

import jax
import jax.numpy as jnp
from jax.experimental import pallas as pl

_K = 16
_TILE = 512


def _dg(a, b, dims):
    return jax.lax.dot_general(a, b, (dims, ((), ())),
                               preferred_element_type=jnp.float32)


def _knn_l0_kernel(p1_ref, p2_ref, f1_ref, f2_ref, w0_ref, b0_ref,
                   y_ref, s_ref, ss_ref):
    b = pl.program_id(0)
    i = pl.program_id(1)
    p1 = p1_ref[0]
    p2 = p2_ref[0]
    sq1 = jnp.sum(p1 * p1, axis=1, keepdims=True)
    sq2 = jnp.sum(p2 * p2, axis=0, keepdims=True)
    d2 = sq1 + sq2 - 2.0 * _dg(p1, p2, ((1,), (0,)))

    d2s = d2
    m = None
    for _ in range(_K):
        m = jnp.min(d2s, axis=0, keepdims=True)
        d2s = jnp.where(d2s <= m, jnp.float32(jnp.inf), d2s)
    mask = d2 <= m
    dist = jnp.sqrt(jnp.maximum(d2, 0.0))
    wr = jnp.where(mask, 1.0 / (dist + 1e-10), 0.0)
    w = wr / jnp.sum(wr, axis=0, keepdims=True)

    newf = _dg(f1_ref[0], w, ((1,), (0,)))
    f2 = f2_ref[0]
    w0 = w0_ref[...]
    c = newf.shape[0]
    y = (_dg(w0[:, :c], newf, ((1,), (0,)))
         + _dg(w0[:, c:], f2, ((1,), (0,)))
         + b0_ref[...])
    y_ref[0] = y

    @pl.when(jnp.logical_and(b == 0, i == 0))
    def _():
        s_ref[...] = jnp.zeros_like(s_ref)
        ss_ref[...] = jnp.zeros_like(ss_ref)

    s_ref[...] += jnp.sum(y, axis=1, keepdims=True)
    ss_ref[...] += jnp.sum(y * y, axis=1, keepdims=True)


def _bn_mlp_kernel(y_ref, s_ref, ss_ref, g_ref, be_ref, w_ref, b_ref,
                   o_ref, s2_ref, ss2_ref, *, ntot):
    b = pl.program_id(0)
    i = pl.program_id(1)
    mean = s_ref[...] / ntot
    var = ss_ref[...] / ntot - mean * mean
    x = (y_ref[0] - mean) * jax.lax.rsqrt(var + 1e-3) * g_ref[...] + be_ref[...]
    h = jnp.maximum(x, 0.0)
    y2 = _dg(w_ref[...], h, ((1,), (0,))) + b_ref[...]

    o_ref[0] = y2

    @pl.when(jnp.logical_and(b == 0, i == 0))
    def _():
        s2_ref[...] = jnp.zeros_like(s2_ref)
        ss2_ref[...] = jnp.zeros_like(ss2_ref)

    s2_ref[...] += jnp.sum(y2, axis=1, keepdims=True)
    ss2_ref[...] += jnp.sum(y2 * y2, axis=1, keepdims=True)


def _bn_out_kernel(y_ref, s_ref, ss_ref, g_ref, be_ref, o_ref, *, ntot):
    mean = s_ref[...] / ntot
    var = ss_ref[...] / ntot - mean * mean
    x = (y_ref[0] - mean) * jax.lax.rsqrt(var + 1e-3) * g_ref[...] + be_ref[...]
    o_ref[0] = jnp.maximum(x, 0.0)


def kernel(points1, points2, features1, features2,
           W0, b0, g0, beta0, W1, b1, g1, beta1, W2, b2, g2, beta2):
    f32 = jnp.float32
    B, _, N1 = points1.shape
    N2 = points2.shape[2]
    C = features1.shape[1]
    OUT0 = W0.shape[0]
    OUT1 = W1.shape[0]
    OUT2 = W2.shape[0]
    T = min(_TILE, N2)
    M = B * N2

    p1t = jnp.concatenate(
        [points1, jnp.zeros((B, 8 - points1.shape[1], N1), f32)],
        axis=1).transpose(0, 2, 1)
    p2p = jnp.concatenate(
        [points2, jnp.zeros((B, 8 - points2.shape[1], N2), f32)],
        axis=1)

    col = lambda v: v.reshape(-1, 1)

    y0, s0, ss0 = pl.pallas_call(
        _knn_l0_kernel,
        grid=(B, N2 // T),
        in_specs=[
            pl.BlockSpec((1, N1, 8), lambda b, i: (b, 0, 0)),
            pl.BlockSpec((1, 8, T), lambda b, i: (b, 0, i)),
            pl.BlockSpec((1, C, N1), lambda b, i: (b, 0, 0)),
            pl.BlockSpec((1, C, T), lambda b, i: (b, 0, i)),
            pl.BlockSpec((OUT0, 2 * C), lambda b, i: (0, 0)),
            pl.BlockSpec((OUT0, 1), lambda b, i: (0, 0)),
        ],
        out_specs=[
            pl.BlockSpec((1, OUT0, T), lambda b, i: (b, 0, i)),
            pl.BlockSpec((OUT0, 1), lambda b, i: (0, 0)),
            pl.BlockSpec((OUT0, 1), lambda b, i: (0, 0)),
        ],
        out_shape=[
            jax.ShapeDtypeStruct((B, OUT0, N2), f32),
            jax.ShapeDtypeStruct((OUT0, 1), f32),
            jax.ShapeDtypeStruct((OUT0, 1), f32),
        ],
    )(p1t, p2p, features1, features2, W0, col(b0))

    def _layer(y, s, ss, g, be, W, bias, cin, cout):
        return pl.pallas_call(
            functools.partial(_bn_mlp_kernel, ntot=float(M)),
            grid=(B, N2 // T),
            in_specs=[
                pl.BlockSpec((1, cin, T), lambda b, i: (b, 0, i)),
                pl.BlockSpec((cin, 1), lambda b, i: (0, 0)),
                pl.BlockSpec((cin, 1), lambda b, i: (0, 0)),
                pl.BlockSpec((cin, 1), lambda b, i: (0, 0)),
                pl.BlockSpec((cin, 1), lambda b, i: (0, 0)),
                pl.BlockSpec((cout, cin), lambda b, i: (0, 0)),
                pl.BlockSpec((cout, 1), lambda b, i: (0, 0)),
            ],
            out_specs=[
                pl.BlockSpec((1, cout, T), lambda b, i: (b, 0, i)),
                pl.BlockSpec((cout, 1), lambda b, i: (0, 0)),
                pl.BlockSpec((cout, 1), lambda b, i: (0, 0)),
            ],
            out_shape=[
                jax.ShapeDtypeStruct((B, cout, N2), f32),
                jax.ShapeDtypeStruct((cout, 1), f32),
                jax.ShapeDtypeStruct((cout, 1), f32),
            ],
        )(y, s, ss, col(g), col(be), W, col(bias))

    y1, s1, ss1 = _layer(y0, s0, ss0, g0, beta0, W1, b1, OUT0, OUT1)
    y2, s2, ss2 = _layer(y1, s1, ss1, g1, beta1, W2, b2, OUT1, OUT2)

    (out,) = pl.pallas_call(
        functools.partial(_bn_out_kernel, ntot=float(M)),
        grid=(B, N2 // T),
        in_specs=[
            pl.BlockSpec((1, OUT2, T), lambda b, i: (b, 0, i)),
            pl.BlockSpec((OUT2, 1), lambda b, i: (0, 0)),
            pl.BlockSpec((OUT2, 1), lambda b, i: (0, 0)),
            pl.BlockSpec((OUT2, 1), lambda b, i: (0, 0)),
            pl.BlockSpec((OUT2, 1), lambda b, i: (0, 0)),
        ],
        out_specs=[pl.BlockSpec((1, OUT2, T), lambda b, i: (b, 0, i))],
        out_shape=[jax.ShapeDtypeStruct((B, OUT2, N2), f32)],
    )(y2, s2, ss2, col(g2), col(beta2))

    return out
